# baseline (speedup 1.0000x reference)
import jax
import jax.numpy as jnp
from jax import lax
from jax.experimental import pallas as pl
from jax.experimental.pallas import tpu as pltpu
from jax.experimental.pallas import tpu_sc as plsc

N = 10000
E = 160000
F = 32
NC = 2
NS = 16
NW = NC * NS
CH = 128
PER_W = 5120
EP = PER_W * NW
NPT = 320
NP = NW * NPT
TAB = NPT + 16
CAP = 6144
SCH = 4096
_GD = lax.GatherDimensionNumbers(offset_dims=(), collapsed_slice_dims=(0,),
                                 start_index_map=(0,))
_SC_PARAMS = pltpu.CompilerParams(needs_layout_passes=False)


def _bcast_lane0(vec):
    idx = jnp.zeros((16, 1), jnp.int32)
    return lax.gather(vec, idx, _GD, (1,),
                      mode=lax.GatherScatterMode.PROMISE_IN_BOUNDS)


def _bucket_body(src_hbm, dst_hbm, a0_hbm, a1_hbm,
                 bsrc_hbm, bdst_hbm, ba0_hbm, ba1_hbm,
                 src_v, dst_v, a0_v, a1_v, ssrc, sdst, sa0, sa1,
                 sem0, sem1, sem2, sem3):
    cid = lax.axis_index("c")
    sid = lax.axis_index("s")
    wid = cid * NS + sid
    lo = wid * NPT

    zi = jnp.zeros((16,), jnp.int32)
    zf = jnp.zeros((16,), jnp.float32)
    dummy = jnp.full((16,), NPT, jnp.int32)

    def init_body(i, c):
        sl = pl.ds(i * 16, 16)
        ssrc[sl] = zi
        sdst[sl] = dummy
        sa0[sl] = zf
        sa1[sl] = zf
        return c

    lax.fori_loop(0, CAP // 16, init_body, 0)

    def chunk_body(c, ptr):
        base = c * SCH
        d0 = pltpu.async_copy(src_hbm.at[pl.ds(base, SCH)], src_v, sem0)
        d1 = pltpu.async_copy(dst_hbm.at[pl.ds(base, SCH)], dst_v, sem1)
        d2 = pltpu.async_copy(a0_hbm.at[pl.ds(base, SCH)], a0_v, sem2)
        d3 = pltpu.async_copy(a1_hbm.at[pl.ds(base, SCH)], a1_v, sem3)
        d0.wait(); d1.wait(); d2.wait(); d3.wait()

        def group_body(g, ptr2):
            sl = pl.ds(g * 16, 16)
            dv = dst_v[sl]
            m = (dv >= lo) & (dv < lo + NPT)
            pu = jnp.minimum(ptr2, CAP - 16)
            psl = pl.ds(pu, 16)
            plsc.store_compressed(ssrc.at[psl], src_v[sl], mask=m)
            plsc.store_compressed(sdst.at[psl], dv - lo, mask=m)
            plsc.store_compressed(sa0.at[psl], a0_v[sl], mask=m)
            plsc.store_compressed(sa1.at[psl], a1_v[sl], mask=m)
            return ptr2 + jnp.sum(m.astype(jnp.int32))

        return lax.fori_loop(0, SCH // 16, group_body, ptr)

    lax.fori_loop(0, EP // SCH, chunk_body, jnp.int32(0))
    pltpu.sync_copy(ssrc, bsrc_hbm.at[wid])
    pltpu.sync_copy(sdst, bdst_hbm.at[wid])
    pltpu.sync_copy(sa0, ba0_hbm.at[wid])
    pltpu.sync_copy(sa1, ba1_hbm.at[wid])


def _bucket(src_p, dst_p, a0_p, a1_p):
    mesh = plsc.VectorSubcoreMesh(core_axis_name="c", subcore_axis_name="s")
    return pl.kernel(
        _bucket_body,
        out_type=(jax.ShapeDtypeStruct((NW, CAP), jnp.int32),
                  jax.ShapeDtypeStruct((NW, CAP), jnp.int32),
                  jax.ShapeDtypeStruct((NW, CAP), jnp.float32),
                  jax.ShapeDtypeStruct((NW, CAP), jnp.float32)),
        mesh=mesh,
        compiler_params=_SC_PARAMS,
        scratch_types=[
            pltpu.VMEM((SCH,), jnp.int32),
            pltpu.VMEM((SCH,), jnp.int32),
            pltpu.VMEM((SCH,), jnp.float32),
            pltpu.VMEM((SCH,), jnp.float32),
            pltpu.VMEM((CAP,), jnp.int32),
            pltpu.VMEM((CAP,), jnp.int32),
            pltpu.VMEM((CAP,), jnp.float32),
            pltpu.VMEM((CAP,), jnp.float32),
            pltpu.SemaphoreType.DMA,
            pltpu.SemaphoreType.DMA,
            pltpu.SemaphoreType.DMA,
            pltpu.SemaphoreType.DMA,
        ],
    )(src_p, dst_p, a0_p, a1_p)


NCH = CAP // CH
NSP = 10112
RSP = NSP // NS


def _edge_body(y_hbm, bsrc_hbm, bdst_hbm, ba0_hbm, ba1_hbm, out_hbm,
               ssrc, sdstl, sa0, sa1, rows_a, rows_b, msg_v, tab_v,
               sem_a, sem_b):
    cid = lax.axis_index("c")
    sid = lax.axis_index("s")
    wid = cid * NS + sid
    lanes = lax.iota(jnp.int32, 16)
    zf = jnp.zeros((16,), jnp.float32)

    d0 = pltpu.async_copy(bsrc_hbm.at[wid], ssrc, sem_a)
    d1 = pltpu.async_copy(bdst_hbm.at[wid], sdstl, sem_b)
    d2 = pltpu.async_copy(ba0_hbm.at[wid], sa0.at[pl.ds(0, CAP)], sem_a)
    d3 = pltpu.async_copy(ba1_hbm.at[wid], sa1.at[pl.ds(0, CAP)], sem_b)

    def zero_body(i, c):
        tab_v[pl.ds(i * 16, 16)] = zf
        return c

    lax.fori_loop(0, TAB * F // 16, zero_body, 0)
    d0.wait(); d1.wait(); d2.wait(); d3.wait()

    pltpu.async_copy(y_hbm.at[ssrc.at[pl.ds(0, CH)]], rows_a, sem_a)

    def _process(c, rows_cur, sem_cur, rows_nxt, sem_nxt):
        cb = c * CH
        nb = jnp.minimum(c + 1, NCH - 1) * CH
        pltpu.async_copy(y_hbm.at[ssrc.at[pl.ds(nb, CH)]], rows_nxt, sem_nxt)
        pltpu.make_async_copy(y_hbm.at[ssrc.at[pl.ds(cb, CH)]],
                              rows_cur, sem_cur).wait()

        def edge_body(e, c2):
            a0 = _bcast_lane0(sa0[pl.ds(cb + e, 16)])
            a1 = _bcast_lane0(sa1[pl.ds(cb + e, 16)])
            for hh in range(2):
                y0 = rows_cur[e, pl.ds(hh * 16, 16)]
                y1 = rows_cur[e, pl.ds(F + hh * 16, 16)]
                yb = rows_cur[e, pl.ds(2 * F + hh * 16, 16)]
                msg_v[pl.ds(e * F + hh * 16, 16)] = a0 * y0 + a1 * y1 + yb
            return c2

        lax.fori_loop(0, CH, edge_body, 0)

        def acc_body(g, c2):
            gb = g * 16
            mrow = sdstl[pl.ds(cb + gb, 16)] * F
            mb = (gb + lanes) * F
            for o in range(F):
                v = plsc.load_gather(msg_v, [mb + o])
                plsc.addupdate_scatter(tab_v, [mrow + o], v)
            return c2

        lax.fori_loop(0, CH // 16, acc_body, 0)

    def chunk_body(c, carry):
        @pl.when(c % 2 == 0)
        def _():
            _process(c, rows_a, sem_a, rows_b, sem_b)

        @pl.when(c % 2 == 1)
        def _():
            _process(c, rows_b, sem_b, rows_a, sem_a)

        return carry

    lax.fori_loop(0, NCH, chunk_body, 0)
    pltpu.make_async_copy(y_hbm.at[ssrc.at[pl.ds((NCH - 1) * CH, CH)]],
                          rows_a, sem_a).wait()
    pltpu.sync_copy(tab_v.at[pl.ds(0, NPT * F)],
                    out_hbm.at[pl.ds(wid * NPT * F, NPT * F)])


def _edge_pass(y, bsrc, bdst, ba0, ba1):
    mesh = plsc.VectorSubcoreMesh(core_axis_name="c", subcore_axis_name="s")
    return pl.kernel(
        _edge_body,
        out_type=jax.ShapeDtypeStruct((NP * F,), jnp.float32),
        mesh=mesh,
        compiler_params=_SC_PARAMS,
        scratch_types=[
            pltpu.VMEM((CAP,), jnp.int32),
            pltpu.VMEM((CAP,), jnp.int32),
            pltpu.VMEM((CAP + 16,), jnp.float32),
            pltpu.VMEM((CAP + 16,), jnp.float32),
            pltpu.VMEM((CH, 4 * F), jnp.float32),
            pltpu.VMEM((CH, 4 * F), jnp.float32),
            pltpu.VMEM((CH * F,), jnp.float32),
            pltpu.VMEM((TAB * F,), jnp.float32),
            pltpu.SemaphoreType.DMA,
            pltpu.SemaphoreType.DMA,
        ],
    )(y, bsrc, bdst, ba0, ba1)


def _prep_body(x_ref, wcat_ref, y_ref):
    y_ref[...] = jnp.dot(x_ref[...], wcat_ref[...],
                         preferred_element_type=jnp.float32,
                         precision=lax.Precision.HIGHEST)


def _tc_prep(x, wcat):
    return pl.pallas_call(
        _prep_body,
        out_shape=jax.ShapeDtypeStruct((N, 4 * F), jnp.float32),
    )(x, wcat)


def _combine_body(agg_ref, h_ref, root_ref, bias_ref, wcat_ref,
                  h_ref_out, y_ref_out):
    hn = agg_ref[...] + jnp.dot(h_ref[...], root_ref[...],
                                preferred_element_type=jnp.float32)
    hn = jnp.maximum(hn + bias_ref[...][None, :], 0.0)
    h_ref_out[...] = hn
    y_ref_out[...] = jnp.dot(hn, wcat_ref[...],
                             preferred_element_type=jnp.float32,
                             precision=lax.Precision.HIGHEST)


def _tc_combine(agg, h, root, bias, wcat):
    return pl.pallas_call(
        _combine_body,
        out_shape=(jax.ShapeDtypeStruct((N, F), jnp.float32),
                   jax.ShapeDtypeStruct((N, 4 * F), jnp.float32)),
    )(agg, h, root, bias, wcat)


def _final_body(agg_ref, h_ref, root_ref, bias_ref,
                fc1_wt_ref, fc1_b_ref, fc2_wt_ref, fc2_b_ref, out_ref):
    hn = agg_ref[...] + jnp.dot(h_ref[...], root_ref[...],
                                preferred_element_type=jnp.float32)
    hn = jnp.maximum(hn + bias_ref[...][None, :], 0.0)
    h4 = jnp.maximum(jnp.dot(hn, fc1_wt_ref[...],
                             preferred_element_type=jnp.float32)
                     + fc1_b_ref[...][None, :], 0.0)
    out_ref[...] = jnp.dot(h4, fc2_wt_ref[...],
                           preferred_element_type=jnp.float32) \
        + fc2_b_ref[...][None, :]


def _tc_final(agg, h, root, bias, fc1_wt, fc1_b, fc2_wt, fc2_b):
    return pl.pallas_call(
        _final_body,
        out_shape=jax.ShapeDtypeStruct((N, 1), jnp.float32),
    )(agg, h, root, bias, fc1_wt, fc1_b, fc2_wt, fc2_b)


def _wcat(w, b, in_ch):
    w0 = w[:, 0].reshape(in_ch, F)
    w1 = w[:, 1].reshape(in_ch, F)
    bm = b.reshape(in_ch, F)
    zz = jnp.zeros((in_ch, F), jnp.float32)
    return jnp.concatenate([w0, w1, bm, zz], axis=1)


def kernel(x, edge_index, edge_attr, w1, b1, root1, bias1, w2, b2, root2,
           bias2, w3, b3, root3, bias3, fc1_w, fc1_b, fc2_w, fc2_b):
    src = edge_index[0].astype(jnp.int32)
    dst = edge_index[1].astype(jnp.int32)
    pad = EP - E
    src_p = jnp.concatenate([src, jnp.zeros((pad,), jnp.int32)])
    dst_p = jnp.concatenate([dst, jnp.full((pad,), NP, jnp.int32)])
    azero = jnp.zeros((pad,), jnp.float32)
    a0_p = jnp.concatenate([edge_attr[:, 0].astype(jnp.float32), azero])
    a1_p = jnp.concatenate([edge_attr[:, 1].astype(jnp.float32), azero])

    bsrc, bdst, ba0, ba1 = _bucket(src_p, dst_p, a0_p, a1_p)

    wcat1 = _wcat(w1, b1, 2)
    wcat2 = _wcat(w2, b2, F)
    wcat3 = _wcat(w3, b3, F)

    def _padr(y):
        return jnp.pad(y, ((0, NSP - N), (0, 0)))

    y1 = _tc_prep(x, wcat1)
    agg1 = _edge_pass(_padr(y1), bsrc, bdst, ba0, ba1).reshape(NP, F)[:N]
    h1, y2 = _tc_combine(agg1, x, root1, bias1, wcat2)
    agg2 = _edge_pass(_padr(y2), bsrc, bdst, ba0, ba1).reshape(NP, F)[:N]
    h2, y3 = _tc_combine(agg2, h1, root2, bias2, wcat3)
    agg3 = _edge_pass(_padr(y3), bsrc, bdst, ba0, ba1).reshape(NP, F)[:N]
    out = _tc_final(agg3, h2, root3, bias3, fc1_w.T, fc1_b, fc2_w.T, fc2_b)
    return out


# speedup vs baseline: 1.0016x; 1.0016x over previous
import jax
import jax.numpy as jnp
from jax import lax
from jax.experimental import pallas as pl
from jax.experimental.pallas import tpu as pltpu
from jax.experimental.pallas import tpu_sc as plsc

N = 10000
E = 160000
F = 32
NC = 2
NS = 16
NW = NC * NS
CH = 128
PER_W = 5120
EP = PER_W * NW
NPT = 320
NP = NW * NPT
TAB = NPT + 16
CAP = 6144
SCH = 4096
_GD = lax.GatherDimensionNumbers(offset_dims=(), collapsed_slice_dims=(0,),
                                 start_index_map=(0,))
_SC_PARAMS = pltpu.CompilerParams(needs_layout_passes=False)


def _bcast_lane0(vec):
    idx = jnp.zeros((16, 1), jnp.int32)
    return lax.gather(vec, idx, _GD, (1,),
                      mode=lax.GatherScatterMode.PROMISE_IN_BOUNDS)


def _bucket_body(src_hbm, dst_hbm, a0_hbm, a1_hbm,
                 bsrc_hbm, bdst_hbm, ba0_hbm, ba1_hbm,
                 src_v, dst_v, a0_v, a1_v, ssrc, sdst, sa0, sa1,
                 sem0, sem1, sem2, sem3):
    cid = lax.axis_index("c")
    sid = lax.axis_index("s")
    wid = cid * NS + sid
    lo = wid * NPT

    zi = jnp.zeros((16,), jnp.int32)
    zf = jnp.zeros((16,), jnp.float32)
    dummy = jnp.full((16,), NPT, jnp.int32)

    def init_body(i, c):
        sl = pl.ds(i * 16, 16)
        ssrc[sl] = zi
        sdst[sl] = dummy
        sa0[sl] = zf
        sa1[sl] = zf
        return c

    lax.fori_loop(0, CAP // 16, init_body, 0)

    def chunk_body(c, ptr):
        base = c * SCH
        d0 = pltpu.async_copy(src_hbm.at[pl.ds(base, SCH)], src_v, sem0)
        d1 = pltpu.async_copy(dst_hbm.at[pl.ds(base, SCH)], dst_v, sem1)
        d2 = pltpu.async_copy(a0_hbm.at[pl.ds(base, SCH)], a0_v, sem2)
        d3 = pltpu.async_copy(a1_hbm.at[pl.ds(base, SCH)], a1_v, sem3)
        d0.wait(); d1.wait(); d2.wait(); d3.wait()

        def group_body(g, ptr2):
            sl = pl.ds(g * 16, 16)
            dv = dst_v[sl]
            m = (dv >= lo) & (dv < lo + NPT)
            pu = jnp.minimum(ptr2, CAP - 16)
            psl = pl.ds(pu, 16)
            plsc.store_compressed(ssrc.at[psl], src_v[sl], mask=m)
            plsc.store_compressed(sdst.at[psl], dv - lo, mask=m)
            plsc.store_compressed(sa0.at[psl], a0_v[sl], mask=m)
            plsc.store_compressed(sa1.at[psl], a1_v[sl], mask=m)
            return ptr2 + jnp.sum(m.astype(jnp.int32))

        return lax.fori_loop(0, SCH // 16, group_body, ptr)

    lax.fori_loop(0, EP // SCH, chunk_body, jnp.int32(0))
    pltpu.sync_copy(ssrc, bsrc_hbm.at[wid])
    pltpu.sync_copy(sdst, bdst_hbm.at[wid])
    pltpu.sync_copy(sa0, ba0_hbm.at[wid])
    pltpu.sync_copy(sa1, ba1_hbm.at[wid])


def _bucket(src_p, dst_p, a0_p, a1_p):
    mesh = plsc.VectorSubcoreMesh(core_axis_name="c", subcore_axis_name="s")
    return pl.kernel(
        _bucket_body,
        out_type=(jax.ShapeDtypeStruct((NW, CAP), jnp.int32),
                  jax.ShapeDtypeStruct((NW, CAP), jnp.int32),
                  jax.ShapeDtypeStruct((NW, CAP), jnp.float32),
                  jax.ShapeDtypeStruct((NW, CAP), jnp.float32)),
        mesh=mesh,
        compiler_params=_SC_PARAMS,
        scratch_types=[
            pltpu.VMEM((SCH,), jnp.int32),
            pltpu.VMEM((SCH,), jnp.int32),
            pltpu.VMEM((SCH,), jnp.float32),
            pltpu.VMEM((SCH,), jnp.float32),
            pltpu.VMEM((CAP,), jnp.int32),
            pltpu.VMEM((CAP,), jnp.int32),
            pltpu.VMEM((CAP,), jnp.float32),
            pltpu.VMEM((CAP,), jnp.float32),
            pltpu.SemaphoreType.DMA,
            pltpu.SemaphoreType.DMA,
            pltpu.SemaphoreType.DMA,
            pltpu.SemaphoreType.DMA,
        ],
    )(src_p, dst_p, a0_p, a1_p)


NCH = CAP // CH
NSP = 10112
RSP = NSP // NS


def _edge_body(y_hbm, bsrc_hbm, bdst_hbm, ba0_hbm, ba1_hbm, out_hbm,
               ssrc, sdstl, sa0, sa1, rows_a, rows_b, msg_v, tab_v,
               sem_a, sem_b):
    cid = lax.axis_index("c")
    sid = lax.axis_index("s")
    wid = cid * NS + sid
    lanes = lax.iota(jnp.int32, 16)
    zf = jnp.zeros((16,), jnp.float32)

    d0 = pltpu.async_copy(bsrc_hbm.at[wid], ssrc, sem_a)
    d1 = pltpu.async_copy(bdst_hbm.at[wid], sdstl, sem_b)
    d2 = pltpu.async_copy(ba0_hbm.at[wid], sa0.at[pl.ds(0, CAP)], sem_a)
    d3 = pltpu.async_copy(ba1_hbm.at[wid], sa1.at[pl.ds(0, CAP)], sem_b)

    def zero_body(i, c):
        tab_v[pl.ds(i * 16, 16)] = zf
        return c

    lax.fori_loop(0, TAB * F // 16, zero_body, 0)
    d0.wait(); d1.wait(); d2.wait(); d3.wait()

    pltpu.async_copy(y_hbm.at[ssrc.at[pl.ds(0, CH)]], rows_a, sem_a)

    def _process(c, rows_cur, sem_cur, rows_nxt, sem_nxt):
        cb = c * CH
        nb = jnp.minimum(c + 1, NCH - 1) * CH
        pltpu.async_copy(y_hbm.at[ssrc.at[pl.ds(nb, CH)]], rows_nxt, sem_nxt)
        pltpu.make_async_copy(y_hbm.at[ssrc.at[pl.ds(cb, CH)]],
                              rows_cur, sem_cur).wait()

        def edge_body(e, c2):
            a0 = _bcast_lane0(sa0[pl.ds(cb + e, 16)])
            a1 = _bcast_lane0(sa1[pl.ds(cb + e, 16)])
            for hh in range(2):
                y0 = rows_cur[e, pl.ds(hh * 16, 16)]
                y1 = rows_cur[e, pl.ds(F + hh * 16, 16)]
                yb = rows_cur[e, pl.ds(2 * F + hh * 16, 16)]
                msg_v[pl.ds(e * F + hh * 16, 16)] = a0 * y0 + a1 * y1 + yb
            return c2

        lax.fori_loop(0, CH, edge_body, 0)

        def acc_body(g, c2):
            gb = g * 16
            mrow = sdstl[pl.ds(cb + gb, 16)] * F
            mb = (gb + lanes) * F
            for o in range(F):
                v = plsc.load_gather(msg_v, [mb + o])
                plsc.addupdate_scatter(tab_v, [mrow + o], v)
            return c2

        lax.fori_loop(0, CH // 16, acc_body, 0)

    def chunk_body(c, carry):
        @pl.when(c % 2 == 0)
        def _():
            _process(c, rows_a, sem_a, rows_b, sem_b)

        @pl.when(c % 2 == 1)
        def _():
            _process(c, rows_b, sem_b, rows_a, sem_a)

        return carry

    lax.fori_loop(0, NCH, chunk_body, 0)
    pltpu.make_async_copy(y_hbm.at[ssrc.at[pl.ds((NCH - 1) * CH, CH)]],
                          rows_a, sem_a).wait()
    pltpu.sync_copy(tab_v.at[pl.ds(0, NPT * F)],
                    out_hbm.at[pl.ds(wid * NPT * F, NPT * F)])


def _edge_pass(y, bsrc, bdst, ba0, ba1):
    mesh = plsc.VectorSubcoreMesh(core_axis_name="c", subcore_axis_name="s")
    return pl.kernel(
        _edge_body,
        out_type=jax.ShapeDtypeStruct((NP * F,), jnp.float32),
        mesh=mesh,
        compiler_params=_SC_PARAMS,
        scratch_types=[
            pltpu.VMEM((CAP,), jnp.int32),
            pltpu.VMEM((CAP,), jnp.int32),
            pltpu.VMEM((CAP + 16,), jnp.float32),
            pltpu.VMEM((CAP + 16,), jnp.float32),
            pltpu.VMEM((CH, 4 * F), jnp.float32),
            pltpu.VMEM((CH, 4 * F), jnp.float32),
            pltpu.VMEM((CH * F,), jnp.float32),
            pltpu.VMEM((TAB * F,), jnp.float32),
            pltpu.SemaphoreType.DMA,
            pltpu.SemaphoreType.DMA,
        ],
    )(y, bsrc, bdst, ba0, ba1)


def _prep_body(x_ref, wcat_ref, y_ref):
    y_ref[...] = jnp.dot(x_ref[...], wcat_ref[...],
                         preferred_element_type=jnp.float32,
                         precision=lax.Precision.HIGHEST)


def _tc_prep(x, wcat):
    return pl.pallas_call(
        _prep_body,
        out_shape=jax.ShapeDtypeStruct((N, 4 * F), jnp.float32),
    )(x, wcat)


def _combine_body(agg_ref, h_ref, root_ref, bias_ref, wcat_ref,
                  h_ref_out, y_ref_out):
    hn = agg_ref[...] + jnp.dot(h_ref[...], root_ref[...],
                                preferred_element_type=jnp.float32)
    hn = jnp.maximum(hn + bias_ref[...][None, :], 0.0)
    h_ref_out[...] = hn
    y_ref_out[...] = jnp.dot(hn, wcat_ref[...],
                             preferred_element_type=jnp.float32,
                             precision=lax.Precision.HIGHEST)


def _tc_combine(agg, h, root, bias, wcat):
    return pl.pallas_call(
        _combine_body,
        out_shape=(jax.ShapeDtypeStruct((N, F), jnp.float32),
                   jax.ShapeDtypeStruct((N, 4 * F), jnp.float32)),
    )(agg, h, root, bias, wcat)


def _final_body(agg_ref, h_ref, root_ref, bias_ref,
                fc1_wt_ref, fc1_b_ref, fc2_wt_ref, fc2_b_ref, out_ref):
    hn = agg_ref[...] + jnp.dot(h_ref[...], root_ref[...],
                                preferred_element_type=jnp.float32)
    hn = jnp.maximum(hn + bias_ref[...][None, :], 0.0)
    h4 = jnp.maximum(jnp.dot(hn, fc1_wt_ref[...],
                             preferred_element_type=jnp.float32)
                     + fc1_b_ref[...][None, :], 0.0)
    out_ref[...] = jnp.dot(h4, fc2_wt_ref[...],
                           preferred_element_type=jnp.float32) \
        + fc2_b_ref[...][None, :]


def _tc_final(agg, h, root, bias, fc1_wt, fc1_b, fc2_wt, fc2_b):
    return pl.pallas_call(
        _final_body,
        out_shape=jax.ShapeDtypeStruct((N, 1), jnp.float32),
    )(agg, h, root, bias, fc1_wt, fc1_b, fc2_wt, fc2_b)


def _wcat(w, b, in_ch):
    w0 = w[:, 0].reshape(in_ch, F)
    w1 = w[:, 1].reshape(in_ch, F)
    bm = b.reshape(in_ch, F)
    zz = jnp.zeros((in_ch, F), jnp.float32)
    return jnp.concatenate([w0, w1, bm, zz], axis=1)


def kernel(x, edge_index, edge_attr, w1, b1, root1, bias1, w2, b2, root2,
           bias2, w3, b3, root3, bias3, fc1_w, fc1_b, fc2_w, fc2_b):
    src = edge_index[0].astype(jnp.int32)
    dst = edge_index[1].astype(jnp.int32)
    pad = EP - E
    src_p = jnp.concatenate([src, jnp.zeros((pad,), jnp.int32)])
    dst_p = jnp.concatenate([dst, jnp.full((pad,), NP, jnp.int32)])
    azero = jnp.zeros((pad,), jnp.float32)
    a0_p = jnp.concatenate([edge_attr[:, 0].astype(jnp.float32), azero])
    a1_p = jnp.concatenate([edge_attr[:, 1].astype(jnp.float32), azero])

    bsrc, bdst, ba0, ba1 = _bucket(src_p, dst_p, a0_p, a1_p)

    wcat1 = _wcat(w1, b1, 2)
    wcat2 = _wcat(w2, b2, F)
    wcat3 = _wcat(w3, b3, F)

    y1 = _tc_prep(x, wcat1)
    agg1 = _edge_pass(y1, bsrc, bdst, ba0, ba1).reshape(NP, F)[:N]
    h1, y2 = _tc_combine(agg1, x, root1, bias1, wcat2)
    agg2 = _edge_pass(y2, bsrc, bdst, ba0, ba1).reshape(NP, F)[:N]
    h2, y3 = _tc_combine(agg2, h1, root2, bias2, wcat3)
    agg3 = _edge_pass(y3, bsrc, bdst, ba0, ba1).reshape(NP, F)[:N]
    out = _tc_final(agg3, h2, root3, bias3, fc1_w.T, fc1_b, fc2_w.T, fc2_b)
    return out
